# Initial kernel scaffold; baseline (speedup 1.0000x reference)
import jax
import jax.numpy as jnp
from jax.experimental import pallas as pl


def kernel(inputs, categ_bias, output_layer_bias):
    raise NotImplementedError("write your pallas kernel here")


# speedup vs baseline: 1.0174x; 1.0174x over previous
import functools

import jax
import jax.numpy as jnp
from jax import lax
from jax.experimental import pallas as pl
from jax.experimental.pallas import tpu as pltpu
from jax.experimental.pallas import tpu_sc as plsc

BATCH = 16384
NC = 2
NS = 16
L = 16
NW = NC * NS
B_PER_W = BATCH // NW


def _gather_body(idx_hbm, table_hbm, bias_hbm, out_hbm, idx_v, rows_v, bias_v, sem):
    wid = lax.axis_index("s") * NC + lax.axis_index("c")
    base = wid * B_PER_W
    pltpu.sync_copy(idx_hbm.at[pl.ds(base, B_PER_W)], idx_v)
    pltpu.async_copy(table_hbm.at[idx_v], rows_v, sem).wait()
    pltpu.sync_copy(bias_hbm, bias_v)
    b = bias_v[...]

    def step(i, carry):
        rows_v[pl.ds(i * L, L)] = rows_v[pl.ds(i * L, L)] + b
        return carry

    lax.fori_loop(0, B_PER_W // L, step, 0)
    pltpu.sync_copy(rows_v, out_hbm.at[pl.ds(base, B_PER_W)])


@jax.jit
def kernel(inputs, categ_bias, output_layer_bias):
    idx = inputs[:, 0].astype(jnp.int32)
    table = categ_bias.reshape(-1)
    bias16 = jnp.broadcast_to(output_layer_bias.reshape(1), (L,))

    mesh = plsc.VectorSubcoreMesh(core_axis_name="c", subcore_axis_name="s")
    run = pl.kernel(
        _gather_body,
        mesh=mesh,
        out_type=jax.ShapeDtypeStruct((BATCH,), jnp.float32),
        scratch_types=[
            pltpu.VMEM((B_PER_W,), jnp.int32),
            pltpu.VMEM((B_PER_W,), jnp.float32),
            pltpu.VMEM((L,), jnp.float32),
            pltpu.SemaphoreType.DMA,
        ],
    )
    out = run(idx, table, bias16)
    return out.reshape(BATCH, 1)


# speedup vs baseline: 2.2020x; 2.1644x over previous
import functools

import jax
import jax.numpy as jnp
from jax import lax
from jax.experimental import pallas as pl
from jax.experimental.pallas import tpu as pltpu
from jax.experimental.pallas import tpu_sc as plsc

BATCH = 16384
NC = 2
NS = 16
L = 16
NW = NC * NS
B_PER_W = BATCH // NW
TABLE_ROWS = 1000000
TABLE_PAD = 448


def _gather_body(idx_hbm, table_hbm, bias_hbm, out_hbm, idx_v, rows_v, bias_v, sem):
    wid = lax.axis_index("s") * NC + lax.axis_index("c")
    base = wid * B_PER_W
    pltpu.sync_copy(idx_hbm.at[pl.ds(base, B_PER_W)], idx_v)
    pltpu.async_copy(table_hbm.at[idx_v], rows_v, sem).wait()
    pltpu.sync_copy(bias_hbm, bias_v)
    b = bias_v[...]

    def step(i, carry):
        rows_v[pl.ds(i * L, L)] = rows_v[pl.ds(i * L, L)] + b
        return carry

    lax.fori_loop(0, B_PER_W // L, step, 0)
    pltpu.sync_copy(rows_v, out_hbm.at[pl.ds(base, B_PER_W)])


@jax.jit
def kernel(inputs, categ_bias, output_layer_bias):
    idx = inputs[:, 0].astype(jnp.int32)
    table = jnp.pad(categ_bias, ((0, TABLE_PAD), (0, 0))).reshape(-1)
    bias16 = jnp.broadcast_to(output_layer_bias.reshape(1), (L,))

    mesh = plsc.VectorSubcoreMesh(core_axis_name="c", subcore_axis_name="s")
    run = pl.kernel(
        _gather_body,
        mesh=mesh,
        out_type=jax.ShapeDtypeStruct((BATCH,), jnp.float32),
        scratch_types=[
            pltpu.VMEM((B_PER_W,), jnp.int32),
            pltpu.VMEM((B_PER_W,), jnp.float32),
            pltpu.VMEM((L,), jnp.float32),
            pltpu.SemaphoreType.DMA,
        ],
    )
    out = run(idx, table, bias16)
    return out.reshape(BATCH, 1)
